# baseline (speedup 1.0000x reference)
import functools

import jax
import jax.numpy as jnp
from jax import lax
from jax.experimental import pallas as pl
from jax.experimental.pallas import tpu as pltpu
from jax.experimental.pallas import tpu_sc as plsc

_NUM_CORES = 2
_NUM_SUBCORES = 16
_NUM_WORKERS = _NUM_CORES * _NUM_SUBCORES

_V_TILE = 2048

_LANES = 128
_SUBL = 8
_NMAIN = (100000 // _LANES) * _LANES
_GSTRIDE = ((_NMAIN // _LANES) + 1) * _LANES * _SUBL


@functools.lru_cache(maxsize=None)
def _make_flatten(vocab, nrows, row0, interpret=False):
    nmain = _NMAIN
    nvreg = nmain // _LANES

    def body(in_ref, out_ref):
        x = in_ref[...]
        main = (
            x[:, :nmain]
            .reshape(_SUBL, nvreg, _LANES)
            .transpose(1, 0, 2)
            .reshape(nmain * _SUBL)
        )
        out_ref[pl.ds(0, nmain * _SUBL)] = main
        side = jnp.pad(
            x[:, nmain:], ((0, 0), (0, _LANES - (vocab - nmain)))
        ).reshape(_SUBL * _LANES)
        out_ref[pl.ds(nmain * _SUBL, _SUBL * _LANES)] = side

    g0 = row0 // _SUBL
    return pl.pallas_call(
        body,
        grid=(nrows // _SUBL,),
        in_specs=[pl.BlockSpec((_SUBL, vocab), lambda j: (j + g0, 0))],
        out_specs=pl.BlockSpec((_GSTRIDE,), lambda j: (j,)),
        out_shape=jax.ShapeDtypeStruct(((nrows // _SUBL) * _GSTRIDE,), jnp.float32),
        interpret=interpret,
    )


@functools.lru_cache(maxsize=None)
def _make_gather_t(vocab, embed, batch):
    assert embed % _NUM_WORKERS == 0
    k_per_w = embed // _NUM_WORKERS
    n_chunks = batch // 128
    assert batch % 128 == 0
    mesh = plsc.VectorSubcoreMesh(core_axis_name="c", subcore_axis_name="s")

    def body(table_hbm, idx_hbm, out_hbm, idx_v, fidx_v, rows_v, sem):
        wid = lax.axis_index("s") * _NUM_CORES + lax.axis_index("c")
        k0 = wid * k_per_w
        pltpu.sync_copy(idx_hbm, idx_v)
        for kk in range(k_per_w):
            k = k0 + kk
            base = (k // _SUBL) * _GSTRIDE + (k % _SUBL) * _LANES
            for i in range(batch // 16):
                iv = idx_v[pl.ds(i * 16, 16)]
                main_f = ((iv >> 7) << 10) + (iv & 127)
                side_f = _NMAIN * _SUBL + (iv - _NMAIN)
                fidx_v[kk, pl.ds(i * 16, 16)] = base + jnp.where(
                    iv < _NMAIN, main_f, side_f
                )
        for kk in range(k_per_w):
            for c in range(n_chunks):
                pltpu.async_copy(
                    table_hbm.at[fidx_v.at[kk, pl.ds(c * 128, 128)]],
                    rows_v.at[kk, pl.ds(c * 128, 128)],
                    sem,
                )
        for kk in range(k_per_w):
            for c in range(n_chunks):
                pltpu.make_async_copy(
                    table_hbm.at[fidx_v.at[kk, pl.ds(c * 128, 128)]],
                    rows_v.at[kk, pl.ds(c * 128, 128)],
                    sem,
                ).wait()
        for kk in range(k_per_w):
            pltpu.sync_copy(rows_v.at[kk], out_hbm.at[k0 + kk])

    return pl.kernel(
        body,
        out_type=jax.ShapeDtypeStruct((embed, batch), jnp.float32),
        mesh=mesh,
        scratch_types=[
            pltpu.VMEM((batch,), jnp.int32),
            pltpu.VMEM((k_per_w, batch), jnp.int32),
            pltpu.VMEM((k_per_w, batch), jnp.float32),
            pltpu.SemaphoreType.DMA,
        ],
        compiler_params=pltpu.CompilerParams(use_tc_tiling_on_sc=False),
    )


def _proj_body(wt_ref, xta_ref, xtb_ref, b_ref, out_ref):
    acc = lax.dot_general(
        wt_ref[...],
        jnp.concatenate([xta_ref[...], xtb_ref[...]], axis=0),
        dimension_numbers=(((0,), (0,)), ((), ())),
        preferred_element_type=jnp.float32,
    )
    ones = jnp.ones((1, acc.shape[1]), jnp.float32)
    out_ref[...] = acc + lax.dot_general(
        b_ref[...],
        ones,
        dimension_numbers=(((0,), (0,)), ((), ())),
        preferred_element_type=jnp.float32,
    )


@functools.lru_cache(maxsize=None)
def _make_proj(batch, embed, vocab, interpret=False):
    grid = (pl.cdiv(vocab, _V_TILE),)
    return pl.pallas_call(
        _proj_body,
        grid=grid,
        in_specs=[
            pl.BlockSpec((embed, _V_TILE), lambda j: (0, j)),
            pl.BlockSpec((embed // 2, batch), lambda j: (0, 0)),
            pl.BlockSpec((embed // 2, batch), lambda j: (0, 0)),
            pl.BlockSpec((1, _V_TILE), lambda j: (0, j)),
        ],
        out_specs=pl.BlockSpec((_V_TILE, batch), lambda j: (j, 0)),
        out_shape=jax.ShapeDtypeStruct((vocab, batch), jnp.float32),
        interpret=interpret,
    )


def kernel(input_ids, token_embedding, head_w, head_b):
    vocab, embed = token_embedding.shape
    (batch,) = input_ids.shape
    idx = input_ids.astype(jnp.int32)
    table_t = token_embedding.T
    half = embed // 2
    flat_a = _make_flatten(vocab, half, 0)(table_t)
    xt_a = _make_gather_t(vocab, half, batch)(flat_a, idx)
    flat_b = _make_flatten(vocab, half, half)(table_t)
    xt_b = _make_gather_t(vocab, half, batch)(flat_b, idx)
    proj = _make_proj(batch, embed, vocab)
    out_t = proj(
        head_w.T,
        xt_a,
        xt_b,
        head_b.reshape(1, vocab),
    )
    return out_t.T


# speedup vs baseline: 1.0196x; 1.0196x over previous
import functools

import jax
import jax.numpy as jnp
from jax import lax
from jax.experimental import pallas as pl
from jax.experimental.pallas import tpu as pltpu
from jax.experimental.pallas import tpu_sc as plsc

_NUM_CORES = 2
_NUM_SUBCORES = 16
_NUM_WORKERS = _NUM_CORES * _NUM_SUBCORES

_V_TILE = 4096

_LANES = 128
_SUBL = 8
_NMAIN = (100000 // _LANES) * _LANES
_GSTRIDE = ((_NMAIN // _LANES) + 1) * _LANES * _SUBL


@functools.lru_cache(maxsize=None)
def _make_flatten(vocab, nrows, row0, interpret=False):
    nmain = _NMAIN
    nvreg = nmain // _LANES

    def body(in_ref, out_ref):
        x = in_ref[...]
        main = (
            x[:, :nmain]
            .reshape(_SUBL, nvreg, _LANES)
            .transpose(1, 0, 2)
            .reshape(nmain * _SUBL)
        )
        out_ref[pl.ds(0, nmain * _SUBL)] = main
        side = jnp.pad(
            x[:, nmain:], ((0, 0), (0, _LANES - (vocab - nmain)))
        ).reshape(_SUBL * _LANES)
        out_ref[pl.ds(nmain * _SUBL, _SUBL * _LANES)] = side

    g0 = row0 // _SUBL
    return pl.pallas_call(
        body,
        grid=(nrows // _SUBL,),
        in_specs=[pl.BlockSpec((_SUBL, vocab), lambda j: (j + g0, 0))],
        out_specs=pl.BlockSpec((_GSTRIDE,), lambda j: (j,)),
        out_shape=jax.ShapeDtypeStruct(((nrows // _SUBL) * _GSTRIDE,), jnp.float32),
        interpret=interpret,
    )


@functools.lru_cache(maxsize=None)
def _make_gather_t(vocab, embed, batch):
    assert embed % _NUM_WORKERS == 0
    k_per_w = embed // _NUM_WORKERS
    n_chunks = batch // 128
    assert batch % 128 == 0
    mesh = plsc.VectorSubcoreMesh(core_axis_name="c", subcore_axis_name="s")

    def body(table_hbm, idx_hbm, out_hbm, idx_v, fidx_v, rows_v, sem):
        wid = lax.axis_index("s") * _NUM_CORES + lax.axis_index("c")
        k0 = wid * k_per_w
        pltpu.sync_copy(idx_hbm, idx_v)
        for kk in range(k_per_w):
            k = k0 + kk
            base = (k // _SUBL) * _GSTRIDE + (k % _SUBL) * _LANES
            for i in range(batch // 16):
                iv = idx_v[pl.ds(i * 16, 16)]
                main_f = ((iv >> 7) << 10) + (iv & 127)
                side_f = _NMAIN * _SUBL + (iv - _NMAIN)
                fidx_v[kk, pl.ds(i * 16, 16)] = base + jnp.where(
                    iv < _NMAIN, main_f, side_f
                )
        for kk in range(k_per_w):
            for c in range(n_chunks):
                pltpu.async_copy(
                    table_hbm.at[fidx_v.at[kk, pl.ds(c * 128, 128)]],
                    rows_v.at[kk, pl.ds(c * 128, 128)],
                    sem,
                )
        for kk in range(k_per_w):
            for c in range(n_chunks):
                pltpu.make_async_copy(
                    table_hbm.at[fidx_v.at[kk, pl.ds(c * 128, 128)]],
                    rows_v.at[kk, pl.ds(c * 128, 128)],
                    sem,
                ).wait()
        for kk in range(k_per_w):
            pltpu.sync_copy(rows_v.at[kk], out_hbm.at[k0 + kk])

    return pl.kernel(
        body,
        out_type=jax.ShapeDtypeStruct((embed, batch), jnp.float32),
        mesh=mesh,
        scratch_types=[
            pltpu.VMEM((batch,), jnp.int32),
            pltpu.VMEM((k_per_w, batch), jnp.int32),
            pltpu.VMEM((k_per_w, batch), jnp.float32),
            pltpu.SemaphoreType.DMA,
        ],
        compiler_params=pltpu.CompilerParams(use_tc_tiling_on_sc=False),
    )


def _proj_body(wt_ref, xt_ref, b_ref, out_ref):
    acc = lax.dot_general(
        wt_ref[...],
        xt_ref[...],
        dimension_numbers=(((0,), (0,)), ((), ())),
        preferred_element_type=jnp.float32,
    )
    ones = jnp.ones((1, acc.shape[1]), jnp.float32)
    out_ref[...] = acc + lax.dot_general(
        b_ref[...],
        ones,
        dimension_numbers=(((0,), (0,)), ((), ())),
        preferred_element_type=jnp.float32,
    )


@functools.lru_cache(maxsize=None)
def _make_proj(batch, embed, vocab, v_tile=_V_TILE, interpret=False):
    grid = (pl.cdiv(vocab, v_tile),)
    return pl.pallas_call(
        _proj_body,
        grid=grid,
        in_specs=[
            pl.BlockSpec((embed, v_tile), lambda j: (0, j)),
            pl.BlockSpec((embed, batch), lambda j: (0, 0)),
            pl.BlockSpec((1, v_tile), lambda j: (0, j)),
        ],
        out_specs=pl.BlockSpec((v_tile, batch), lambda j: (j, 0)),
        out_shape=jax.ShapeDtypeStruct((vocab, batch), jnp.float32),
        interpret=interpret,
    )


def kernel(input_ids, token_embedding, head_w, head_b):
    vocab, embed = token_embedding.shape
    (batch,) = input_ids.shape
    idx = input_ids.astype(jnp.int32)
    table_flat = _make_flatten(vocab, embed, 0)(token_embedding.T)
    xt = _make_gather_t(vocab, embed, batch)(table_flat, idx)
    proj = _make_proj(batch, embed, vocab)
    out_t = proj(
        head_w.T,
        xt,
        head_b.reshape(1, vocab),
    )
    return out_t.T
